# baseline (speedup 1.0000x reference)
import functools

import jax
import jax.numpy as jnp
from jax import lax
from jax.experimental import pallas as pl
from jax.experimental.pallas import tpu as pltpu
from jax.experimental.pallas import tpu_sc as plsc

_NC = 2
_NS = 16
_NW = _NC * _NS
_SUP = 512
_L = 16


def _splat(x, dtype=jnp.int32):
    return jnp.full((_L,), 0, dtype) + x


def _lane(v, k):
    return lax.squeeze(lax.slice(v, [k], [k + 1]), [0])


@functools.lru_cache(maxsize=None)
def _make_gather(B: int, D: int, V: int):
    n_full = V // _SUP
    n_sup = n_full + 1
    n_pairs = (n_sup + 2 * _NW - 1) // (2 * _NW)
    mesh = plsc.VectorSubcoreMesh(core_axis_name="c", subcore_axis_name="s")

    @functools.partial(
        pl.kernel,
        mesh=mesh,
        out_type=jax.ShapeDtypeStruct((B + _NW, 128), jnp.float32),
        scratch_types=[
            pltpu.VMEM((B,), jnp.int32),
            pltpu.VMEM((B,), jnp.int32),
            pltpu.VMEM((B,), jnp.int32),
            pltpu.VMEM((D, _SUP), jnp.float32),
            pltpu.VMEM((D, _SUP), jnp.float32),
            pltpu.VMEM((1024,), jnp.int32),
            pltpu.VMEM((32,), jnp.int32),
            pltpu.VMEM((32,), jnp.int32),
            pltpu.VMEM((112, 128), jnp.float32),
            pltpu.VMEM((1, 112), jnp.int32),
            pltpu.SMEM((8,), jnp.int32),
            pltpu.SemaphoreType.DMA,
            pltpu.SemaphoreType.DMA,
        ],
        compiler_params=pltpu.CompilerParams(
            use_tc_tiling_on_sc=True, needs_layout_passes=False
        ),
    )
    def gather_kernel(idx_hbm, table_hbm, tail_hbm, out_hbm, idx_v, blist,
                      blist2, buf0, buf1, occ, pend_r, pend_b, rows_v,
                      bchunk, sc, sem, sem2):
        wid = lax.axis_index("s") * _NC + lax.axis_index("c")
        trash = B + wid
        iota = lax.iota(jnp.int32, _L)
        zero = _splat(0)

        pltpu.sync_copy(idx_hbm, idx_v)
        sc[0] = 0
        sc[1] = 0
        for t8 in range(7):
            bchunk[0, pl.ds(t8 * _L, _L)] = _splat(trash)
        for z in range(1024 // _L):
            occ[pl.ds(z * _L, _L)] = zero

        def sub_conds(s):
            t = lax.shift_right_logical(s, 5)
            flags = occ[pl.ds(t * _L, _L)]
            return [_lane(flags, k) > 0 for k in range(4)]

        def fetch(s, buf):
            conds = sub_conds(s)

            @pl.when(s < n_full)
            def _():
                for k in range(4):
                    @pl.when(conds[k])
                    def _(k=k):
                        pltpu.async_copy(
                            table_hbm.at[:, pl.ds(s * _SUP + k * 128, 128)],
                            buf.at[:, pl.ds(k * 128, 128)],
                            sem,
                        )

            @pl.when(s == n_full)
            def _():
                @pl.when(conds[0])
                def _():
                    pltpu.async_copy(
                        tail_hbm.at[:, pl.ds(0, 128)],
                        buf.at[:, pl.ds(0, 128)],
                        sem,
                    )

        def drain(s, buf):
            conds = sub_conds(s)
            for k in range(4):
                cond = conds[k]
                if k > 0:
                    cond = jnp.logical_and(cond, s < n_full)

                @pl.when(cond)
                def _(k=k):
                    pltpu.make_async_copy(
                        table_hbm.at[:, pl.ds(0, 128)],
                        buf.at[:, pl.ds(k * 128, 128)],
                        sem,
                    ).wait()

        def scan_body(i, cnt):
            v = idx_v[pl.ds(i * _L, _L)]
            j = lax.shift_right_logical(v, 9)
            m = lax.bitwise_and(j, _splat(_NW - 1)) == _splat(wid)
            pos = cnt + plsc.cumsum(jnp.where(m, 1, 0).astype(jnp.int32)) - 1
            plsc.store_scatter(blist, [pos], iota + i * _L, mask=m)
            slot = lax.shift_left(lax.shift_right_logical(j, 5), 4) + (
                lax.bitwise_and(lax.shift_right_logical(v, 7), _splat(3))
            )
            plsc.store_scatter(occ, [slot], _splat(1), mask=m)
            return _lane(pos, _L - 1) + 1

        count = lax.fori_loop(0, B // _L, scan_body, 0)
        ngrp = lax.shift_right_logical(count + (_L - 1), 4)

        fetch(wid, buf0)

        @pl.when(wid + _NW < n_sup)
        def _():
            fetch(wid + _NW, buf1)

        def radix_pass(bit, src, dst):
            def body(i, carry):
                lo, hi = carry
                b16 = src[pl.ds(i * _L, _L)]
                valid = (iota + i * _L) < _splat(count)
                bv = jnp.where(valid, b16, 0)
                r16 = plsc.load_gather(idx_v, [bv])
                t16 = lax.shift_right_logical(r16, 14)
                isone = (
                    lax.bitwise_and(t16, _splat(1 << bit)) > zero
                )
                m0 = jnp.logical_and(valid, jnp.logical_not(isone))
                m1 = jnp.logical_and(valid, isone)
                c0 = plsc.cumsum(jnp.where(m0, 1, 0).astype(jnp.int32))
                c1 = plsc.cumsum(jnp.where(m1, 1, 0).astype(jnp.int32))
                plsc.store_scatter(dst, [lo + c0 - 1], b16, mask=m0)
                plsc.store_scatter(dst, [hi + c1 - 1], b16, mask=m1)
                return (lo + _lane(c0, _L - 1), hi + _lane(c1, _L - 1))

            return body

        def zcnt_body(i, zc):
            b16 = blist[pl.ds(i * _L, _L)]
            valid = (iota + i * _L) < _splat(count)
            bv = jnp.where(valid, b16, 0)
            r16 = plsc.load_gather(idx_v, [bv])
            t16 = lax.shift_right_logical(r16, 14)
            out = []
            for bit in range(6):
                z = jnp.logical_and(
                    valid,
                    lax.bitwise_and(t16, _splat(1 << bit)) == zero,
                )
                out.append(zc[bit] + plsc.all_reduce_population_count(z)[0])
            return tuple(out)

        zcnts = lax.fori_loop(0, ngrp, zcnt_body, (0, 0, 0, 0, 0, 0))
        src, dst = blist, blist2
        for bit in range(6):
            lax.fori_loop(0, ngrp, radix_pass(bit, src, dst),
                          (0, zcnts[bit]))
            src, dst = dst, src
        slist = src

        def flush():
            pltpu.async_copy(rows_v, out_hbm.at[bchunk.at[0]], sem2).wait()
            for t8 in range(7):
                bchunk[0, pl.ds(t8 * _L, _L)] = _splat(trash)
            sc[0] = 0

        def extract(buf, r16, b16, m, nh):
            @pl.when(sc[0] > 112 - _L)
            def _():
                flush()

            w0 = sc[0]
            col16 = lax.bitwise_and(r16, _splat(_SUP - 1))
            slot16 = w0 + iota

            def dgrp(dg, carry):
                for du in range(16):
                    d = dg * 16 + du
                    vals = plsc.load_gather(buf, [_splat(d), col16], mask=m)
                    plsc.store_scatter(
                        rows_v, [slot16, _splat(d)], vals, mask=m
                    )
                return carry

            lax.fori_loop(0, D // 16, dgrp, 0)
            plsc.store_scatter(bchunk, [_splat(0), slot16], b16, mask=m)
            sc[0] = w0 + nh

        def process(s, buf):
            def cond(carry):
                return carry[0]

            def body(carry):
                _, g = carry
                b16 = slist[pl.ds(g * _L, _L)]
                valid = (iota + g * _L) < _splat(count)
                bv = jnp.where(valid, b16, 0)
                r16 = plsc.load_gather(idx_v, [bv])
                j16 = lax.shift_right_logical(r16, 9)
                m = jnp.logical_and(valid, j16 == _splat(s))
                csum = plsc.cumsum(jnp.where(m, 1, 0).astype(jnp.int32))
                nh = _lane(csum, _L - 1)

                @pl.when(nh > 0)
                def _():
                    pc = sc[1]
                    ppos = pc + csum - 1
                    plsc.store_scatter(pend_r, [ppos], r16, mask=m)
                    plsc.store_scatter(pend_b, [ppos], b16, mask=m)
                    sc[1] = pc + nh

                    @pl.when(sc[1] >= _L)
                    def _():
                        pr = pend_r[pl.ds(0, _L)]
                        pb = pend_b[pl.ds(0, _L)]
                        extract(buf, pr, pb, zero < _splat(1), _L)
                        pend_r[pl.ds(0, _L)] = pend_r[pl.ds(_L, _L)]
                        pend_b[pl.ds(0, _L)] = pend_b[pl.ds(_L, _L)]
                        sc[1] = sc[1] - _L

                later = jnp.logical_and(valid, j16 > _splat(s))
                nlater = plsc.all_reduce_population_count(later)[0]
                adv = nlater == 0
                g2 = g + jnp.where(adv, 1, 0)
                more = jnp.logical_and(adv, g2 * _L < count)
                return (more, g2)

            g0 = sc[2]
            res = lax.while_loop(cond, body, (g0 * _L < count, g0))
            sc[2] = res[1]

            @pl.when(sc[1] > 0)
            def _():
                pc = sc[1]
                pr = pend_r[pl.ds(0, _L)]
                pb = pend_b[pl.ds(0, _L)]
                mv = iota < _splat(pc)
                extract(buf, pr, pb, mv, pc)
                sc[1] = 0

        sc[2] = 0
        bufs = (buf0, buf1)

        def pair(p, carry):
            for q in range(2):
                t = 2 * p + q
                s_cur = wid + t * _NW
                s_pre = s_cur + 2 * _NW

                @pl.when(s_cur < n_sup)
                def _(s_cur=s_cur, q=q):
                    drain(s_cur, bufs[q])
                    process(s_cur, bufs[q])

                @pl.when(s_pre < n_sup)
                def _(s_pre=s_pre, q=q):
                    fetch(s_pre, bufs[q])

            return carry

        lax.fori_loop(0, n_pairs, pair, 0)
        flush()

    return gather_kernel


def kernel(input, table):
    s0, s1 = input.shape
    v, d = table.shape
    idx = input.reshape(-1).astype(jnp.int32)
    b = idx.shape[0]
    table_t = table.T
    n_full = v // _SUP
    tail = table_t[:, n_full * _SUP:]
    tail_pad = jnp.pad(tail, ((0, 0), (0, _SUP - tail.shape[1])))
    out_raw = _make_gather(b, d, v)(idx, table_t, tail_pad)
    return out_raw[:b, :d].reshape(s0, s1, d)


# speedup vs baseline: 1.0171x; 1.0171x over previous
import functools

import jax
import jax.numpy as jnp
from jax import lax
from jax.experimental import pallas as pl
from jax.experimental.pallas import tpu as pltpu
from jax.experimental.pallas import tpu_sc as plsc

_NC = 2
_NS = 16
_NW = _NC * _NS
_SUP = 512
_L = 16


def _splat(x, dtype=jnp.int32):
    return jnp.full((_L,), 0, dtype) + x


def _lane(v, k):
    return lax.squeeze(lax.slice(v, [k], [k + 1]), [0])


@functools.lru_cache(maxsize=None)
def _make_gather(B: int, D: int, V: int):
    n_full = V // _SUP
    n_sup = n_full + 1
    n_pairs = (n_sup + 2 * _NW - 1) // (2 * _NW)
    mesh = plsc.VectorSubcoreMesh(core_axis_name="c", subcore_axis_name="s")

    @functools.partial(
        pl.kernel,
        mesh=mesh,
        out_type=jax.ShapeDtypeStruct((B + _NW, 128), jnp.float32),
        scratch_types=[
            pltpu.VMEM((B,), jnp.int32),
            pltpu.VMEM((B,), jnp.int32),
            pltpu.VMEM((B,), jnp.int32),
            pltpu.VMEM((D, _SUP), jnp.float32),
            pltpu.VMEM((D, _SUP), jnp.float32),
            pltpu.VMEM((1024,), jnp.int32),
            pltpu.VMEM((32,), jnp.int32),
            pltpu.VMEM((32,), jnp.int32),
            pltpu.VMEM((112, 128), jnp.float32),
            pltpu.VMEM((1, 112), jnp.int32),
            pltpu.SMEM((8,), jnp.int32),
            pltpu.SemaphoreType.DMA,
            pltpu.SemaphoreType.DMA,
        ],
        compiler_params=pltpu.CompilerParams(
            use_tc_tiling_on_sc=True, needs_layout_passes=False
        ),
    )
    def gather_kernel(idx_hbm, table_hbm, tail_hbm, out_hbm, idx_v, blist,
                      blist2, buf0, buf1, occ, pend_r, pend_b, rows_v,
                      bchunk, sc, sem, sem2):
        wid = lax.axis_index("s") * _NC + lax.axis_index("c")
        trash = B + wid
        iota = lax.iota(jnp.int32, _L)
        zero = _splat(0)

        pltpu.sync_copy(idx_hbm, idx_v)
        sc[0] = 0
        sc[1] = 0
        for t8 in range(7):
            bchunk[0, pl.ds(t8 * _L, _L)] = _splat(trash)
        for z in range(1024 // _L):
            occ[pl.ds(z * _L, _L)] = zero

        def sub_conds(s):
            t = lax.shift_right_logical(s, 5)
            flags = occ[pl.ds(t * _L, _L)]
            return [_lane(flags, k) > 0 for k in range(4)]

        def fetch(s, buf):
            conds = sub_conds(s)

            @pl.when(s < n_full)
            def _():
                for k in range(4):
                    @pl.when(conds[k])
                    def _(k=k):
                        pltpu.async_copy(
                            table_hbm.at[:, pl.ds(s * _SUP + k * 128, 128)],
                            buf.at[:, pl.ds(k * 128, 128)],
                            sem,
                        )

            @pl.when(s == n_full)
            def _():
                @pl.when(conds[0])
                def _():
                    pltpu.async_copy(
                        tail_hbm.at[:, pl.ds(0, 128)],
                        buf.at[:, pl.ds(0, 128)],
                        sem,
                    )

        def drain(s, buf):
            conds = sub_conds(s)
            for k in range(4):
                cond = conds[k]
                if k > 0:
                    cond = jnp.logical_and(cond, s < n_full)

                @pl.when(cond)
                def _(k=k):
                    pltpu.make_async_copy(
                        table_hbm.at[:, pl.ds(0, 128)],
                        buf.at[:, pl.ds(k * 128, 128)],
                        sem,
                    ).wait()

        def scan_half(i, cnt):
            v = idx_v[pl.ds(i * _L, _L)]
            j = lax.shift_right_logical(v, 9)
            m = lax.bitwise_and(j, _splat(_NW - 1)) == _splat(wid)
            pos = cnt + plsc.cumsum(jnp.where(m, 1, 0).astype(jnp.int32)) - 1
            plsc.store_scatter(blist, [pos], iota + i * _L, mask=m)
            slot = lax.shift_left(lax.shift_right_logical(j, 5), 4) + (
                lax.bitwise_and(lax.shift_right_logical(v, 7), _splat(3))
            )
            plsc.store_scatter(occ, [slot], _splat(1), mask=m)
            return _lane(pos, _L - 1) + 1

        def scan_body(i2, cnt):
            cnt = scan_half(2 * i2, cnt)
            return scan_half(2 * i2 + 1, cnt)

        count = lax.fori_loop(0, B // _L // 2, scan_body, 0)
        ngrp = lax.shift_right_logical(count + (_L - 1), 4)

        fetch(wid, buf0)

        @pl.when(wid + _NW < n_sup)
        def _():
            fetch(wid + _NW, buf1)

        def radix_pass(bit, src, dst):
            def body(i, carry):
                lo, hi = carry
                b16 = src[pl.ds(i * _L, _L)]
                valid = (iota + i * _L) < _splat(count)
                bv = jnp.where(valid, b16, 0)
                r16 = plsc.load_gather(idx_v, [bv])
                t16 = lax.shift_right_logical(r16, 14)
                isone = (
                    lax.bitwise_and(t16, _splat(1 << bit)) > zero
                )
                m0 = jnp.logical_and(valid, jnp.logical_not(isone))
                m1 = jnp.logical_and(valid, isone)
                c0 = plsc.cumsum(jnp.where(m0, 1, 0).astype(jnp.int32))
                c1 = plsc.cumsum(jnp.where(m1, 1, 0).astype(jnp.int32))
                plsc.store_scatter(dst, [lo + c0 - 1], b16, mask=m0)
                plsc.store_scatter(dst, [hi + c1 - 1], b16, mask=m1)
                return (lo + _lane(c0, _L - 1), hi + _lane(c1, _L - 1))

            return body

        def zcnt_body(i, zc):
            b16 = blist[pl.ds(i * _L, _L)]
            valid = (iota + i * _L) < _splat(count)
            bv = jnp.where(valid, b16, 0)
            r16 = plsc.load_gather(idx_v, [bv])
            t16 = lax.shift_right_logical(r16, 14)
            out = []
            for bit in range(6):
                z = jnp.logical_and(
                    valid,
                    lax.bitwise_and(t16, _splat(1 << bit)) == zero,
                )
                out.append(zc[bit] + plsc.all_reduce_population_count(z)[0])
            return tuple(out)

        zcnts = lax.fori_loop(0, ngrp, zcnt_body, (0, 0, 0, 0, 0, 0))
        src, dst = blist, blist2
        for bit in range(6):
            lax.fori_loop(0, ngrp, radix_pass(bit, src, dst),
                          (0, zcnts[bit]))
            src, dst = dst, src
        slist = src

        def flush():
            pltpu.async_copy(rows_v, out_hbm.at[bchunk.at[0]], sem2).wait()
            for t8 in range(7):
                bchunk[0, pl.ds(t8 * _L, _L)] = _splat(trash)
            sc[0] = 0

        def extract(buf, r16, b16, m, nh):
            @pl.when(sc[0] > 112 - _L)
            def _():
                flush()

            w0 = sc[0]
            col16 = lax.bitwise_and(r16, _splat(_SUP - 1))
            slot16 = w0 + iota
            for d in range(D):
                vals = plsc.load_gather(buf, [_splat(d), col16], mask=m)
                plsc.store_scatter(rows_v, [slot16, _splat(d)], vals, mask=m)
            plsc.store_scatter(bchunk, [_splat(0), slot16], b16, mask=m)
            sc[0] = w0 + nh

        def process(s, buf):
            def cond(carry):
                return carry[0]

            def body(carry):
                _, g = carry
                b16 = slist[pl.ds(g * _L, _L)]
                valid = (iota + g * _L) < _splat(count)
                bv = jnp.where(valid, b16, 0)
                r16 = plsc.load_gather(idx_v, [bv])
                j16 = lax.shift_right_logical(r16, 9)
                m = jnp.logical_and(valid, j16 == _splat(s))
                csum = plsc.cumsum(jnp.where(m, 1, 0).astype(jnp.int32))
                nh = _lane(csum, _L - 1)

                @pl.when(nh > 0)
                def _():
                    pc = sc[1]
                    ppos = pc + csum - 1
                    plsc.store_scatter(pend_r, [ppos], r16, mask=m)
                    plsc.store_scatter(pend_b, [ppos], b16, mask=m)
                    sc[1] = pc + nh

                    @pl.when(sc[1] >= _L)
                    def _():
                        pr = pend_r[pl.ds(0, _L)]
                        pb = pend_b[pl.ds(0, _L)]
                        extract(buf, pr, pb, zero < _splat(1), _L)
                        pend_r[pl.ds(0, _L)] = pend_r[pl.ds(_L, _L)]
                        pend_b[pl.ds(0, _L)] = pend_b[pl.ds(_L, _L)]
                        sc[1] = sc[1] - _L

                later = jnp.logical_and(valid, j16 > _splat(s))
                nlater = plsc.all_reduce_population_count(later)[0]
                adv = nlater == 0
                g2 = g + jnp.where(adv, 1, 0)
                more = jnp.logical_and(adv, g2 * _L < count)
                return (more, g2)

            g0 = sc[2]
            res = lax.while_loop(cond, body, (g0 * _L < count, g0))
            sc[2] = res[1]

            @pl.when(sc[1] > 0)
            def _():
                pc = sc[1]
                pr = pend_r[pl.ds(0, _L)]
                pb = pend_b[pl.ds(0, _L)]
                mv = iota < _splat(pc)
                extract(buf, pr, pb, mv, pc)
                sc[1] = 0

        sc[2] = 0
        bufs = (buf0, buf1)

        def pair(p, carry):
            for q in range(2):
                t = 2 * p + q
                s_cur = wid + t * _NW
                s_pre = s_cur + 2 * _NW

                @pl.when(s_cur < n_sup)
                def _(s_cur=s_cur, q=q):
                    drain(s_cur, bufs[q])
                    process(s_cur, bufs[q])

                @pl.when(s_pre < n_sup)
                def _(s_pre=s_pre, q=q):
                    fetch(s_pre, bufs[q])

            return carry

        lax.fori_loop(0, n_pairs, pair, 0)
        flush()

    return gather_kernel


def kernel(input, table):
    s0, s1 = input.shape
    v, d = table.shape
    idx = input.reshape(-1).astype(jnp.int32)
    b = idx.shape[0]
    table_t = table.T
    n_full = v // _SUP
    tail = table_t[:, n_full * _SUP:]
    tail_pad = jnp.pad(tail, ((0, 0), (0, _SUP - tail.shape[1])))
    out_raw = _make_gather(b, d, v)(idx, table_t, tail_pad)
    return out_raw[:b, :d].reshape(s0, s1, d)
